# baseline (speedup 1.0000x reference)
import jax
import jax.numpy as jnp
from jax import lax
from jax.experimental import pallas as pl
from jax.experimental.pallas import tpu as pltpu
from jax.experimental.pallas import tpu_sc as plsc

_N_CLS = 1000
_ROWS = 16384
_SCALE = 1.0 / (_ROWS * _N_CLS)

_TC_NJ = 800
_SC_J0 = _TC_NJ
_SC_NJ = _N_CLS - _TC_NJ

_BJ = 8
_NSTREAM = 25
_TC_GRID = _TC_NJ // (_BJ * _NSTREAM)

_NC, _NS, _L = 2, 16, 16
_NW = _NC * _NS
_CPW = _ROWS // _NW
_CJ = 8
_NCHS = _SC_NJ // _CJ


def _tc_body(t_ref, *rest):
    p_refs, o_ref = rest[:_NSTREAM], rest[_NSTREAM]
    i = pl.program_id(0)
    t = t_ref[...]
    partial = jnp.float32(0.0)
    for s, p_ref in enumerate(p_refs):
        base = (_NSTREAM * i + s) * _BJ
        j = lax.broadcasted_iota(jnp.int32, (_BJ, _ROWS), 0) + base
        w = jnp.abs(j.astype(jnp.float32) - t) * jnp.float32(_SCALE)
        partial += jnp.sum(w * p_ref[...])

    @pl.when(i == 0)
    def _init():
        o_ref[0, 0] = 0.0

    o_ref[0, 0] += partial


def _chunk_cost(buf, t_v, j0f, acc):
    def col_chunk(cc, acc):
        t16 = t_v[pl.ds(cc * _L, _L)]
        jv = jnp.full((_L,), j0f, jnp.float32)
        acc0 = acc
        acc1 = jnp.zeros((_L,), jnp.float32)
        for r in range(_CJ):
            p16 = buf[r, pl.ds(cc * _L, _L)]
            if r % 2 == 0:
                acc0 = acc0 + jnp.abs(jv - t16) * p16
            else:
                acc1 = acc1 + jnp.abs(jv - t16) * p16
            jv = jv + jnp.float32(1.0)
        return acc0 + acc1

    return lax.fori_loop(0, _CPW // _L, col_chunk, acc)


def _sc_body(pt_hbm, t_hbm, out_hbm, t_v, buf0, buf1, acc_v, sem0, sem1):
    cid = lax.axis_index("c")
    sid = lax.axis_index("s")
    wid = sid * _NC + cid
    col0 = wid * _CPW

    pltpu.sync_copy(t_hbm.at[pl.ds(col0, _CPW)], t_v)
    pltpu.async_copy(
        pt_hbm.at[pl.ds(_SC_J0, _CJ), pl.ds(col0, _CPW)], buf0, sem0)
    pltpu.async_copy(
        pt_hbm.at[pl.ds(_SC_J0 + _CJ, _CJ), pl.ds(col0, _CPW)], buf1, sem1)

    def pair(g2, acc):
        c0 = 2 * g2
        j0 = _SC_J0 + c0 * _CJ
        pltpu.make_async_copy(
            pt_hbm.at[pl.ds(j0, _CJ), pl.ds(col0, _CPW)], buf0, sem0).wait()
        acc = _chunk_cost(buf0, t_v, j0.astype(jnp.float32), acc)

        @pl.when(c0 + 2 < _NCHS)
        def _():
            pltpu.async_copy(
                pt_hbm.at[pl.ds(_SC_J0 + (c0 + 2) * _CJ, _CJ),
                          pl.ds(col0, _CPW)], buf0, sem0)

        j1 = _SC_J0 + (c0 + 1) * _CJ
        pltpu.make_async_copy(
            pt_hbm.at[pl.ds(j1, _CJ), pl.ds(col0, _CPW)], buf1, sem1).wait()
        acc = _chunk_cost(buf1, t_v, j1.astype(jnp.float32), acc)

        @pl.when(c0 + 3 < _NCHS)
        def _():
            pltpu.async_copy(
                pt_hbm.at[pl.ds(_SC_J0 + (c0 + 3) * _CJ, _CJ),
                          pl.ds(col0, _CPW)], buf1, sem1)

        return acc

    acc = lax.fori_loop(0, _NCHS // 2, pair, jnp.zeros((_L,), jnp.float32))
    if _NCHS % 2:
        jt = _SC_J0 + (_NCHS - 1) * _CJ
        pltpu.make_async_copy(
            pt_hbm.at[pl.ds(jt, _CJ), pl.ds(col0, _CPW)], buf0, sem0).wait()
        acc = _chunk_cost(buf0, t_v, jnp.float32(jt), acc)
    acc_v[...] = acc * jnp.float32(_SCALE)
    pltpu.sync_copy(acc_v, out_hbm.at[wid])


def kernel(output_probs, target_class):
    pt = output_probs.T
    t_f = target_class.astype(jnp.float32)

    mesh = plsc.VectorSubcoreMesh(core_axis_name="c", subcore_axis_name="s")
    sc_fn = pl.kernel(
        _sc_body,
        mesh=mesh,
        out_type=jax.ShapeDtypeStruct((_NW, _L), jnp.float32),
        scratch_types=[
            pltpu.VMEM((_CPW,), jnp.float32),
            pltpu.VMEM((_CJ, _CPW), jnp.float32),
            pltpu.VMEM((_CJ, _CPW), jnp.float32),
            pltpu.VMEM((_L,), jnp.float32),
            pltpu.SemaphoreType.DMA,
            pltpu.SemaphoreType.DMA,
        ],
    )
    sc_out = sc_fn(pt, t_f)

    t_row = t_f.reshape(1, _ROWS)
    in_specs = [pl.BlockSpec((1, _ROWS), lambda i: (0, 0))]
    for s in range(_NSTREAM):
        in_specs.append(
            pl.BlockSpec((_BJ, _ROWS), lambda i, s=s: (_NSTREAM * i + s, 0)))
    tc_out = pl.pallas_call(
        _tc_body,
        grid=(_TC_GRID,),
        in_specs=in_specs,
        out_specs=pl.BlockSpec(memory_space=pltpu.SMEM),
        out_shape=jax.ShapeDtypeStruct((1, 1), jnp.float32),
    )(t_row, *([pt] * _NSTREAM))
    return tc_out[0, 0] + jnp.sum(sc_out)


# speedup vs baseline: 1.7554x; 1.7554x over previous
import jax
import jax.numpy as jnp
from jax import lax
from jax.experimental import pallas as pl
from jax.experimental.pallas import tpu as pltpu

_N_CLS = 1000
_ROWS = 16384
_SCALE = 1.0 / (_ROWS * _N_CLS)

_BJ = 8
_NSTREAM = 25
_GRID = _N_CLS // (_BJ * _NSTREAM)


def _tc_body(t_ref, *rest):
    p_refs, o_ref = rest[:_NSTREAM], rest[_NSTREAM]
    i = pl.program_id(0)
    t = t_ref[...]
    partial = jnp.float32(0.0)
    for s, p_ref in enumerate(p_refs):
        base = (_NSTREAM * i + s) * _BJ
        j = lax.broadcasted_iota(jnp.int32, (_BJ, _ROWS), 0) + base
        w = jnp.abs(j.astype(jnp.float32) - t) * jnp.float32(_SCALE)
        partial += jnp.sum(w * p_ref[...])

    @pl.when(i == 0)
    def _init():
        o_ref[0, 0] = 0.0

    o_ref[0, 0] += partial


def kernel(output_probs, target_class):
    pt = output_probs.T
    t_row = target_class.astype(jnp.float32).reshape(1, _ROWS)
    in_specs = [pl.BlockSpec((1, _ROWS), lambda i: (0, 0))]
    for s in range(_NSTREAM):
        in_specs.append(
            pl.BlockSpec((_BJ, _ROWS), lambda i, s=s: (_NSTREAM * i + s, 0)))
    out = pl.pallas_call(
        _tc_body,
        grid=(_GRID,),
        in_specs=in_specs,
        out_specs=pl.BlockSpec(memory_space=pltpu.SMEM),
        out_shape=jax.ShapeDtypeStruct((1, 1), jnp.float32),
    )(t_row, *([pt] * _NSTREAM))
    return out[0, 0]
